# Initial kernel scaffold; baseline (speedup 1.0000x reference)
import jax
import jax.numpy as jnp
from jax.experimental import pallas as pl


def kernel(inputs, output_shape, unpool_mat):
    raise NotImplementedError("write your pallas kernel here")


# speedup vs baseline: 1.3954x; 1.3954x over previous
import jax
import jax.numpy as jnp
from jax import lax
from jax.experimental import pallas as pl
from jax.experimental.pallas import tpu as pltpu
from jax.experimental.pallas import tpu_sc as plsc

B_, H_, W_, C_ = 4, 384, 384, 96
FLAT = B_ * H_ * W_ * C_
N = B_ * (H_ // 2) * (W_ // 2) * C_
LGB = 20
BSZ = 1 << LGB
NB = FLAT // BSZ
NC, NS = 2, 16
NW = NC * NS
NPW = N // NW
WIN = 8192
NWIN = NPW // WIN
ROWS_W = NPW // 128
CHUNK = 2048
CHTOT = CHUNK * NS
CAP = N + NB * CHTOT
NBH = NC_BUCKETS = NB // NC

_MESH = plsc.VectorSubcoreMesh(
    core_axis_name="c", subcore_axis_name="s", num_cores=NC, num_subcores=NS)
_PARAMS = pltpu.CompilerParams(needs_layout_passes=False)

LANE = lambda: jnp.arange(16, dtype=jnp.int32)


def _hist_kernel(idx2, hist, idxwin, lhist):
    c = lax.axis_index("c")
    s = lax.axis_index("s")
    w = c * NS + s
    lane = LANE()
    zi = jnp.zeros((16,), jnp.int32)
    ones = jnp.ones((16,), jnp.int32)
    for r in range(64):
        lhist[pl.ds(r * 16, 16)] = zi

    def wbody(j, _):
        pltpu.sync_copy(idx2.at[pl.ds(pl.multiple_of(w * ROWS_W + j * 64, 64), 64)], idxwin)

        def rbody(i, _):
            for cc in range(8):
                v = idxwin[i, pl.ds(cc * 16, 16)]
                a = lax.shift_right_logical(v, LGB) * 16 + lane
                cnt = plsc.load_gather(lhist, [a])
                plsc.store_scatter(lhist, [a], cnt + ones)
            return 0

        lax.fori_loop(0, 64, rbody, 0)
        return 0

    lax.fori_loop(0, NWIN, wbody, 0)
    pltpu.sync_copy(lhist, hist.at[w])


def _part_kernel(idx2, val2, hist, bidx, bval,
                 idxwin, valwin, destb, lowb, histv, totals, bs, gcur,
                 sem1, sem2):
    c = lax.axis_index("c")
    s = lax.axis_index("s")
    w = c * NS + s
    lane = LANE()
    z16 = jnp.zeros((16,), jnp.int32)
    pltpu.sync_copy(hist, histv)

    def dbody(d, _):
        acc = z16
        accb = z16
        for wp in range(NW):
            row = histv[wp, pl.ds(d * 16, 16)]
            acc = acc + row
            m = jnp.where(wp < w, jnp.int32(1), jnp.int32(0))
            accb = accb + row * m
        own = histv[w, pl.ds(d * 16, 16)]
        exl = plsc.cumsum(own) - own
        totals[d] = jnp.sum(acc)
        gcur[pl.ds(d * 16, 16)] = exl + jnp.sum(accb)
        return 0

    lax.fori_loop(0, NB, dbody, 0)

    def bbody(d, carry):
        bs[d] = carry
        t = totals[d]
        cap = jnp.bitwise_and(t + (CHTOT - 1), jnp.int32(~(CHTOT - 1)))
        return carry + cap

    lax.fori_loop(0, NB, bbody, jnp.int32(0))

    def gbody(d, _):
        gcur[pl.ds(d * 16, 16)] = gcur[pl.ds(d * 16, 16)] + bs[d]
        return 0

    lax.fori_loop(0, NB, gbody, 0)

    def wbody(j, _):
        row0 = pl.multiple_of(w * ROWS_W + j * 64, 64)
        pltpu.sync_copy(idx2.at[pl.ds(row0, 64)], idxwin)
        pltpu.sync_copy(val2.at[pl.ds(row0, 64)], valwin)

        def rbody(i, _):
            for cc in range(8):
                v = idxwin[i, pl.ds(cc * 16, 16)]
                a = lax.shift_right_logical(v, LGB) * 16 + lane
                p = plsc.load_gather(gcur, [a])
                plsc.store_scatter(gcur, [a], p + 1)
                destb[i, pl.ds(cc * 16, 16)] = p
                lowb[i, pl.ds(cc * 16, 16)] = jnp.bitwise_and(
                    v, jnp.int32(BSZ - 1))
            return 0

        lax.fori_loop(0, 64, rbody, 0)
        for g in range(4):
            descs = []
            for i in range(16):
                r = g * 16 + i
                descs.append(pltpu.async_copy(
                    lowb.at[r], bidx.at[destb.at[r]], sem1))
                descs.append(pltpu.async_copy(
                    valwin.at[r], bval.at[destb.at[r]], sem2))
            for d_ in descs:
                d_.wait()
        return 0

    lax.fori_loop(0, NWIN, wbody, 0)


def _accum_kernel(bidx2, bval2, hist, out1,
                  histv, totals, bs, zeros, idxch, valch, spmem, semc):
    c = lax.axis_index("c")
    s = lax.axis_index("s")
    lane = LANE()
    z16 = jnp.zeros((16,), jnp.int32)
    zf = jnp.zeros((16,), jnp.float32)
    pltpu.sync_copy(hist, histv)

    def dbody(d, _):
        acc = z16
        for wp in range(NW):
            acc = acc + histv[wp, pl.ds(d * 16, 16)]
        totals[d] = jnp.sum(acc)
        return 0

    lax.fori_loop(0, NB, dbody, 0)

    def bbody(d, carry):
        bs[d] = carry
        t = totals[d]
        cap = jnp.bitwise_and(t + (CHTOT - 1), jnp.int32(~(CHTOT - 1)))
        return carry + cap

    lax.fori_loop(0, NB, bbody, jnp.int32(0))

    for r in range(256):
        zeros[pl.ds(r * 16, 16)] = zf

    def bucket(jb, _):
        b = c * NBH + jb
        cnt = totals[b]
        base = bs[b]
        nch = lax.shift_right_logical(cnt + (CHTOT - 1), 15)
        for t in range(16):
            pltpu.sync_copy(zeros, spmem.at[pl.ds(pl.multiple_of(s * 65536 + t * 4096, 4096), 4096)])
        plsc.subcore_barrier()

        def chunk(t, _):
            loff = (t * NS + s) * CHUNK
            row0 = pl.multiple_of(lax.shift_right_logical(base + loff, 7), 16)
            pltpu.sync_copy(bidx2.at[pl.ds(row0, 16)], idxch)
            pltpu.sync_copy(bval2.at[pl.ds(row0, 16)], valch)

            @pl.when(loff + CHUNK > cnt)
            def _():
                for i in range(16):
                    for cc in range(8):
                        e0 = i * 128 + cc * 16
                        m = (loff + e0 + lane) < cnt
                        iv = idxch[i, pl.ds(cc * 16, 16)]
                        vv = valch[i, pl.ds(cc * 16, 16)]
                        idxch[i, pl.ds(cc * 16, 16)] = jnp.where(
                            m, iv, e0 + lane)
                        valch[i, pl.ds(cc * 16, 16)] = jnp.where(m, vv, 0.0)

            descs = []
            for i in range(16):
                descs.append(pltpu.async_copy(
                    valch.at[i], spmem.at[idxch.at[i]], semc, add=True))
            for d_ in descs:
                d_.wait()
            return 0

        lax.fori_loop(0, nch, chunk, 0)
        plsc.subcore_barrier()
        pltpu.sync_copy(
            spmem.at[pl.ds(pl.multiple_of(s * 65536, 65536), 65536)],
            out1.at[pl.ds(pl.multiple_of(b * BSZ + s * 65536, 65536), 65536)])
        plsc.subcore_barrier()
        return 0

    lax.fori_loop(0, NBH, bucket, 0)


def kernel(inputs, output_shape, unpool_mat):
    del output_shape
    idx2 = unpool_mat.reshape(N // 128, 128)
    val2 = inputs.reshape(N // 128, 128)

    hist = pl.kernel(
        _hist_kernel,
        out_type=jax.ShapeDtypeStruct((NW, 1024), jnp.int32),
        mesh=_MESH,
        compiler_params=_PARAMS,
        scratch_types=[
            pltpu.VMEM((64, 128), jnp.int32),
            pltpu.VMEM((1024,), jnp.int32),
        ],
    )(idx2)

    bidx, bval = pl.kernel(
        _part_kernel,
        out_type=(jax.ShapeDtypeStruct((CAP,), jnp.int32),
                  jax.ShapeDtypeStruct((CAP,), jnp.float32)),
        mesh=_MESH,
        compiler_params=_PARAMS,
        scratch_types=[
            pltpu.VMEM((64, 128), jnp.int32),
            pltpu.VMEM((64, 128), jnp.float32),
            pltpu.VMEM((64, 128), jnp.int32),
            pltpu.VMEM((64, 128), jnp.int32),
            pltpu.VMEM((NW, 1024), jnp.int32),
            pltpu.SMEM((64,), jnp.int32),
            pltpu.SMEM((64,), jnp.int32),
            pltpu.VMEM((1024,), jnp.int32),
            pltpu.SemaphoreType.DMA,
            pltpu.SemaphoreType.DMA,
        ],
    )(idx2, val2, hist)

    out1 = pl.kernel(
        _accum_kernel,
        out_type=jax.ShapeDtypeStruct((FLAT,), jnp.float32),
        mesh=_MESH,
        compiler_params=_PARAMS,
        scratch_types=[
            pltpu.VMEM((NW, 1024), jnp.int32),
            pltpu.SMEM((64,), jnp.int32),
            pltpu.SMEM((64,), jnp.int32),
            pltpu.VMEM((4096,), jnp.float32),
            pltpu.VMEM((16, 128), jnp.int32),
            pltpu.VMEM((16, 128), jnp.float32),
            pltpu.VMEM_SHARED((BSZ,), jnp.float32),
            pltpu.SemaphoreType.DMA,
        ],
    )(bidx.reshape(CAP // 128, 128), bval.reshape(CAP // 128, 128), hist)

    return out1.reshape(B_, H_, W_, C_)
